# baseline (speedup 1.0000x reference)
import functools

import jax
import jax.numpy as jnp
from jax import lax
from jax.experimental import pallas as pl
from jax.experimental.pallas import tpu as pltpu
from jax.experimental.pallas import tpu_sc as plsc

NUM_CORES = 2
NUM_SUBCORES = 16
NUM_WORKERS = NUM_CORES * NUM_SUBCORES
NBUF = 4
BLK = 128
RELAYOUT_BLK = 2048


def _relayout_table(token_table):
  v, e = token_table.shape
  half = RELAYOUT_BLK // 2
  rows_out = RELAYOUT_BLK * e // 128
  assert half * e % 64 == 0 and 2 * e == 128
  t_t = token_table.T
  steps = (v + RELAYOUT_BLK - 1) // RELAYOUT_BLK

  def body(in_ref, out_ref):
    x = in_ref[...]
    out_ref[...] = jnp.concatenate([x[:, :half].T, x[:, half:].T], axis=1)

  return pl.pallas_call(
      body,
      grid=(steps,),
      in_specs=[pl.BlockSpec((e, RELAYOUT_BLK), lambda j: (0, j))],
      out_specs=pl.BlockSpec((rows_out, 128), lambda j: (j, 0)),
      out_shape=jax.ShapeDtypeStruct((steps * rows_out, 128), jnp.float32),
      compiler_params=pltpu.CompilerParams(
          dimension_semantics=("parallel",)),
  )(t_t)


def _permute_idx(idx):
  half = RELAYOUT_BLK // 2
  u = idx % RELAYOUT_BLK
  return idx - u + (u % half) * 2 + u // half


def _build_sc_gather(batch, maxlen, embed):
  rows_total = batch * maxlen
  blocks_total = rows_total // BLK
  assert blocks_total % NUM_WORKERS == 0
  blk_per_w = blocks_total // NUM_WORKERS
  rows_per_w = blk_per_w * BLK
  assert blk_per_w >= NBUF

  mesh = plsc.VectorSubcoreMesh(core_axis_name="c", subcore_axis_name="s")

  row_buf = pltpu.VMEM((BLK, embed), jnp.float32)

  @functools.partial(
      pl.kernel,
      mesh=mesh,
      compiler_params=pltpu.CompilerParams(use_tc_tiling_on_sc=False),
      out_type=jax.ShapeDtypeStruct((rows_total, embed), jnp.float32),
      scratch_types=[
          pltpu.VMEM((rows_per_w,), jnp.int32),
          [row_buf] * NBUF,
          [pltpu.SemaphoreType.DMA] * NBUF,
          [pltpu.SemaphoreType.DMA] * NBUF,
      ],
  )
  def k(table_hbm, idx_hbm, out_hbm, idx_v, bufs, sems_in, sems_out):
    wid = lax.axis_index("s") * NUM_CORES + lax.axis_index("c")
    base = pl.multiple_of(wid * rows_per_w, 8)
    pltpu.sync_copy(idx_hbm.at[pl.ds(base, rows_per_w)], idx_v)

    def issue_gather(i, b):
      return pltpu.async_copy(
          table_hbm.at[idx_v.at[pl.ds(i * BLK, BLK)]], bufs[b], sems_in[b])

    def issue_out(i, b):
      return pltpu.async_copy(
          bufs[b], out_hbm.at[pl.ds(base + i * BLK, BLK)], sems_out[b])

    gather_h = [issue_gather(i, i) for i in range(NBUF)]
    out_h = [None] * NBUF

    for i in range(blk_per_w):
      b = i % NBUF
      gather_h[b].wait()
      out_h[b] = issue_out(i, b)

      ni = i + NBUF - 1
      if NBUF <= ni < blk_per_w:
        nb = ni % NBUF
        out_h[nb].wait()
        gather_h[nb] = issue_gather(ni, nb)

    for b in range(NBUF):
      out_h[(blk_per_w - NBUF + b) % NBUF].wait()

  return k


def _transpose_add(inter, pos_table, batch, maxlen, embed):
  inter3 = inter.reshape(maxlen, batch // 2, 2 * embed)
  pos_rep = jnp.repeat(pos_table.T, 128, axis=1)

  def body(in_ref, pos_ref, out_ref):
    x = in_ref[0]
    pc = pos_ref[:, 0:1]
    out_ref[0] = (
        jnp.concatenate([x[:, :embed].T, x[:, embed:].T], axis=1) + pc)

  return pl.pallas_call(
      body,
      grid=(maxlen,),
      in_specs=[
          pl.BlockSpec((1, batch // 2, 2 * embed), lambda l: (l, 0, 0)),
          pl.BlockSpec((embed, 128), lambda l: (0, l)),
      ],
      out_specs=pl.BlockSpec((1, embed, batch), lambda l: (l, 0, 0)),
      out_shape=jax.ShapeDtypeStruct((maxlen, embed, batch), jnp.float32),
      compiler_params=pltpu.CompilerParams(
          dimension_semantics=("parallel",)),
  )(inter3, pos_rep)


@jax.jit
def kernel(x, token_table, pos_table):
  batch, maxlen = x.shape
  embed = token_table.shape[1]
  x_t = x.T.astype(jnp.int32)
  idx_seq = x_t.reshape(maxlen, 2, batch // 2).swapaxes(1, 2).reshape(-1)
  idx_flat = _permute_idx(idx_seq)
  table_lin = _relayout_table(token_table).reshape(-1, embed)
  k = _build_sc_gather(batch, maxlen, embed)
  inter = k(table_lin, idx_flat)
  out_t = _transpose_add(inter, pos_table, batch, maxlen, embed)
  return out_t.transpose(2, 0, 1)


# speedup vs baseline: 1.3211x; 1.3211x over previous
import functools

import jax
import jax.numpy as jnp
from jax import lax
from jax.experimental import pallas as pl
from jax.experimental.pallas import tpu as pltpu
from jax.experimental.pallas import tpu_sc as plsc

NUM_CORES = 2
NUM_SUBCORES = 16
NUM_WORKERS = NUM_CORES * NUM_SUBCORES
NBUF = 4
BLK = 128
RELAYOUT_BLK = 4096
POS_PER_STEP = 2


def _relayout_table(token_table):
  v, e = token_table.shape
  half = RELAYOUT_BLK // 2
  rows_out = RELAYOUT_BLK * e // 128
  assert half * e % 64 == 0 and 2 * e == 128
  t_t = token_table.T
  steps = (v + RELAYOUT_BLK - 1) // RELAYOUT_BLK

  def body(in_ref, out_ref):
    x = in_ref[...]
    out_ref[...] = jnp.concatenate([x[:, :half].T, x[:, half:].T], axis=1)

  return pl.pallas_call(
      body,
      grid=(steps,),
      in_specs=[pl.BlockSpec((e, RELAYOUT_BLK), lambda j: (0, j))],
      out_specs=pl.BlockSpec((rows_out, 128), lambda j: (j, 0)),
      out_shape=jax.ShapeDtypeStruct((steps * rows_out, 128), jnp.float32),
      compiler_params=pltpu.CompilerParams(
          dimension_semantics=("parallel",)),
  )(t_t)


def _permute_idx(idx):
  half = RELAYOUT_BLK // 2
  u = idx % RELAYOUT_BLK
  return idx - u + (u % half) * 2 + u // half


def _build_sc_gather(batch, maxlen, embed):
  rows_total = batch * maxlen
  blocks_total = rows_total // BLK
  assert blocks_total % NUM_WORKERS == 0
  blk_per_w = blocks_total // NUM_WORKERS
  rows_per_w = blk_per_w * BLK
  assert blk_per_w >= NBUF

  mesh = plsc.VectorSubcoreMesh(core_axis_name="c", subcore_axis_name="s")

  row_buf = pltpu.VMEM((BLK, embed), jnp.float32)

  @functools.partial(
      pl.kernel,
      mesh=mesh,
      compiler_params=pltpu.CompilerParams(use_tc_tiling_on_sc=False),
      out_type=jax.ShapeDtypeStruct((rows_total, embed), jnp.float32),
      scratch_types=[
          pltpu.VMEM((rows_per_w,), jnp.int32),
          [row_buf] * NBUF,
          [pltpu.SemaphoreType.DMA] * NBUF,
          [pltpu.SemaphoreType.DMA] * NBUF,
      ],
  )
  def k(table_hbm, idx_hbm, out_hbm, idx_v, bufs, sems_in, sems_out):
    wid = lax.axis_index("s") * NUM_CORES + lax.axis_index("c")
    base = pl.multiple_of(wid * rows_per_w, 8)
    pltpu.sync_copy(idx_hbm.at[pl.ds(base, rows_per_w)], idx_v)

    def issue_gather(i, b):
      return pltpu.async_copy(
          table_hbm.at[idx_v.at[pl.ds(i * BLK, BLK)]], bufs[b], sems_in[b])

    def issue_out(i, b):
      return pltpu.async_copy(
          bufs[b], out_hbm.at[pl.ds(base + i * BLK, BLK)], sems_out[b])

    gather_h = [issue_gather(i, i) for i in range(NBUF)]
    out_h = [None] * NBUF

    for i in range(blk_per_w):
      b = i % NBUF
      gather_h[b].wait()
      out_h[b] = issue_out(i, b)

      ni = i + NBUF - 1
      if NBUF <= ni < blk_per_w:
        nb = ni % NBUF
        out_h[nb].wait()
        gather_h[nb] = issue_gather(ni, nb)

    for b in range(NBUF):
      out_h[(blk_per_w - NBUF + b) % NBUF].wait()

  return k


def _transpose_add(inter, pos_table, batch, maxlen, embed):
  inter3 = inter.reshape(maxlen, batch // 2, 2 * embed)
  pos_rep = jnp.repeat(pos_table.T, 128, axis=1)

  def body(in_ref, pos_ref, out_ref):
    for j in range(POS_PER_STEP):
      x = in_ref[j]
      pc = pos_ref[:, j * 128:j * 128 + 1]
      out_ref[j] = (
          jnp.concatenate([x[:, :embed].T, x[:, embed:].T], axis=1) + pc)

  return pl.pallas_call(
      body,
      grid=(maxlen // POS_PER_STEP,),
      in_specs=[
          pl.BlockSpec((POS_PER_STEP, batch // 2, 2 * embed),
                       lambda l: (l, 0, 0)),
          pl.BlockSpec((embed, POS_PER_STEP * 128), lambda l: (0, l)),
      ],
      out_specs=pl.BlockSpec((POS_PER_STEP, embed, batch),
                             lambda l: (l, 0, 0)),
      out_shape=jax.ShapeDtypeStruct((maxlen, embed, batch), jnp.float32),
      compiler_params=pltpu.CompilerParams(
          dimension_semantics=("parallel",)),
  )(inter3, pos_rep)


@jax.jit
def kernel(x, token_table, pos_table):
  batch, maxlen = x.shape
  embed = token_table.shape[1]
  x_t = x.T.astype(jnp.int32)
  idx_seq = x_t.reshape(maxlen, 2, batch // 2).swapaxes(1, 2).reshape(-1)
  idx_flat = _permute_idx(idx_seq)
  table_lin = _relayout_table(token_table).reshape(-1, embed)
  k = _build_sc_gather(batch, maxlen, embed)
  inter = k(table_lin, idx_flat)
  out_t = _transpose_add(inter, pos_table, batch, maxlen, embed)
  return out_t.transpose(2, 0, 1)


# speedup vs baseline: 1.5820x; 1.1975x over previous
import functools

import jax
import jax.numpy as jnp
from jax import lax
from jax.experimental import pallas as pl
from jax.experimental.pallas import tpu as pltpu
from jax.experimental.pallas import tpu_sc as plsc

NUM_CORES = 2
NUM_SUBCORES = 16
NUM_WORKERS = NUM_CORES * NUM_SUBCORES
NBUF = 4
BLK = 128
RELAYOUT_BLK = 8192
POS_PER_STEP = 4


def _relayout_table(token_table):
  v, e = token_table.shape
  half = RELAYOUT_BLK // 2
  rows_out = RELAYOUT_BLK * e // 128
  assert half * e % 64 == 0 and 2 * e == 128
  t_t = token_table.T
  steps = (v + RELAYOUT_BLK - 1) // RELAYOUT_BLK

  def body(in_ref, out_ref):
    x = in_ref[...]
    out_ref[...] = jnp.concatenate([x[:, :half].T, x[:, half:].T], axis=1)

  return pl.pallas_call(
      body,
      grid=(steps,),
      in_specs=[pl.BlockSpec((e, RELAYOUT_BLK), lambda j: (0, j))],
      out_specs=pl.BlockSpec((rows_out, 128), lambda j: (j, 0)),
      out_shape=jax.ShapeDtypeStruct((steps * rows_out, 128), jnp.float32),
      compiler_params=pltpu.CompilerParams(
          dimension_semantics=("parallel",)),
  )(t_t)


def _permute_idx(idx):
  half = RELAYOUT_BLK // 2
  u = idx % RELAYOUT_BLK
  return idx - u + (u % half) * 2 + u // half


def _build_sc_gather(batch, maxlen, embed):
  rows_total = batch * maxlen
  blocks_total = rows_total // BLK
  assert blocks_total % NUM_WORKERS == 0
  blk_per_w = blocks_total // NUM_WORKERS
  rows_per_w = blk_per_w * BLK
  assert blk_per_w >= NBUF

  mesh = plsc.VectorSubcoreMesh(core_axis_name="c", subcore_axis_name="s")

  row_buf = pltpu.VMEM((BLK, embed), jnp.float32)

  @functools.partial(
      pl.kernel,
      mesh=mesh,
      compiler_params=pltpu.CompilerParams(use_tc_tiling_on_sc=False),
      out_type=jax.ShapeDtypeStruct((rows_total, embed), jnp.float32),
      scratch_types=[
          pltpu.VMEM((rows_per_w,), jnp.int32),
          [row_buf] * NBUF,
          [pltpu.SemaphoreType.DMA] * NBUF,
          [pltpu.SemaphoreType.DMA] * NBUF,
      ],
  )
  def k(table_hbm, idx_hbm, out_hbm, idx_v, bufs, sems_in, sems_out):
    wid = lax.axis_index("s") * NUM_CORES + lax.axis_index("c")
    base = pl.multiple_of(wid * rows_per_w, 8)
    pltpu.sync_copy(idx_hbm.at[pl.ds(base, rows_per_w)], idx_v)

    def issue_gather(i, b):
      return pltpu.async_copy(
          table_hbm.at[idx_v.at[pl.ds(i * BLK, BLK)]], bufs[b], sems_in[b])

    def issue_out(i, b):
      return pltpu.async_copy(
          bufs[b], out_hbm.at[pl.ds(base + i * BLK, BLK)], sems_out[b])

    gather_h = [issue_gather(i, i) for i in range(NBUF)]
    out_h = [None] * NBUF

    for i in range(blk_per_w):
      b = i % NBUF
      gather_h[b].wait()
      out_h[b] = issue_out(i, b)

      ni = i + NBUF - 1
      if NBUF <= ni < blk_per_w:
        nb = ni % NBUF
        out_h[nb].wait()
        gather_h[nb] = issue_gather(ni, nb)

    for b in range(NBUF):
      out_h[(blk_per_w - NBUF + b) % NBUF].wait()

  return k


def _transpose_add(inter, pos_table, batch, maxlen, embed):
  inter3 = inter.reshape(maxlen, batch // 2, 2 * embed)
  pos_rep = jnp.repeat(pos_table.T, 128, axis=1)

  def body(in_ref, pos_ref, out_ref):
    for j in range(POS_PER_STEP):
      x = in_ref[j]
      pc = pos_ref[:, j * 128:j * 128 + 1]
      out_ref[j] = (
          jnp.concatenate([x[:, :embed].T, x[:, embed:].T], axis=1) + pc)

  return pl.pallas_call(
      body,
      grid=(maxlen // POS_PER_STEP,),
      in_specs=[
          pl.BlockSpec((POS_PER_STEP, batch // 2, 2 * embed),
                       lambda l: (l, 0, 0)),
          pl.BlockSpec((embed, POS_PER_STEP * 128), lambda l: (0, l)),
      ],
      out_specs=pl.BlockSpec((POS_PER_STEP, embed, batch),
                             lambda l: (l, 0, 0)),
      out_shape=jax.ShapeDtypeStruct((maxlen, embed, batch), jnp.float32),
      compiler_params=pltpu.CompilerParams(
          dimension_semantics=("parallel",)),
  )(inter3, pos_rep)


@jax.jit
def kernel(x, token_table, pos_table):
  batch, maxlen = x.shape
  embed = token_table.shape[1]
  x_t = x.T.astype(jnp.int32)
  idx_seq = x_t.reshape(maxlen, 2, batch // 2).swapaxes(1, 2).reshape(-1)
  idx_flat = _permute_idx(idx_seq)
  table_lin = _relayout_table(token_table).reshape(-1, embed)
  k = _build_sc_gather(batch, maxlen, embed)
  inter = k(table_lin, idx_flat)
  out_t = _transpose_add(inter, pos_table, batch, maxlen, embed)
  return out_t.transpose(2, 0, 1)


# speedup vs baseline: 1.7585x; 1.1116x over previous
import functools

import jax
import jax.numpy as jnp
from jax import lax
from jax.experimental import pallas as pl
from jax.experimental.pallas import tpu as pltpu
from jax.experimental.pallas import tpu_sc as plsc

NUM_CORES = 2
NUM_SUBCORES = 16
NUM_WORKERS = NUM_CORES * NUM_SUBCORES
NBUF = 4
BLK = 128
RELAYOUT_BLK = 16384
POS_PER_STEP = 8


def _relayout_table(token_table):
  v, e = token_table.shape
  half = RELAYOUT_BLK // 2
  rows_out = RELAYOUT_BLK * e // 128
  assert half * e % 64 == 0 and 2 * e == 128
  t_t = token_table.T
  steps = (v + RELAYOUT_BLK - 1) // RELAYOUT_BLK

  def body(in_ref, out_ref):
    x = in_ref[...]
    out_ref[...] = jnp.concatenate([x[:, :half].T, x[:, half:].T], axis=1)

  return pl.pallas_call(
      body,
      grid=(steps,),
      in_specs=[pl.BlockSpec((e, RELAYOUT_BLK), lambda j: (0, j))],
      out_specs=pl.BlockSpec((rows_out, 128), lambda j: (j, 0)),
      out_shape=jax.ShapeDtypeStruct((steps * rows_out, 128), jnp.float32),
      compiler_params=pltpu.CompilerParams(
          dimension_semantics=("parallel",)),
  )(t_t)


def _permute_idx(idx):
  half = RELAYOUT_BLK // 2
  u = idx % RELAYOUT_BLK
  return idx - u + (u % half) * 2 + u // half


def _build_sc_gather(batch, maxlen, embed):
  rows_total = batch * maxlen
  blocks_total = rows_total // BLK
  assert blocks_total % NUM_WORKERS == 0
  blk_per_w = blocks_total // NUM_WORKERS
  rows_per_w = blk_per_w * BLK
  assert blk_per_w >= NBUF

  mesh = plsc.VectorSubcoreMesh(core_axis_name="c", subcore_axis_name="s")

  row_buf = pltpu.VMEM((BLK, embed), jnp.float32)

  @functools.partial(
      pl.kernel,
      mesh=mesh,
      compiler_params=pltpu.CompilerParams(use_tc_tiling_on_sc=False),
      out_type=jax.ShapeDtypeStruct((rows_total, embed), jnp.float32),
      scratch_types=[
          pltpu.VMEM((rows_per_w,), jnp.int32),
          [row_buf] * NBUF,
          [pltpu.SemaphoreType.DMA] * NBUF,
          [pltpu.SemaphoreType.DMA] * NBUF,
      ],
  )
  def k(table_hbm, idx_hbm, out_hbm, idx_v, bufs, sems_in, sems_out):
    wid = lax.axis_index("s") * NUM_CORES + lax.axis_index("c")
    base = pl.multiple_of(wid * rows_per_w, 8)
    pltpu.sync_copy(idx_hbm.at[pl.ds(base, rows_per_w)], idx_v)

    def issue_gather(i, b):
      return pltpu.async_copy(
          table_hbm.at[idx_v.at[pl.ds(i * BLK, BLK)]], bufs[b], sems_in[b])

    def issue_out(i, b):
      return pltpu.async_copy(
          bufs[b], out_hbm.at[pl.ds(base + i * BLK, BLK)], sems_out[b])

    gather_h = [issue_gather(i, i) for i in range(NBUF)]
    out_h = [None] * NBUF

    for i in range(blk_per_w):
      b = i % NBUF
      gather_h[b].wait()
      out_h[b] = issue_out(i, b)

      ni = i + NBUF - 1
      if NBUF <= ni < blk_per_w:
        nb = ni % NBUF
        out_h[nb].wait()
        gather_h[nb] = issue_gather(ni, nb)

    for b in range(NBUF):
      out_h[(blk_per_w - NBUF + b) % NBUF].wait()

  return k


def _transpose_add(inter, pos_table, batch, maxlen, embed):
  inter3 = inter.reshape(maxlen, batch // 2, 2 * embed)
  pos_rep = jnp.repeat(pos_table.T, 128, axis=1)

  def body(in_ref, pos_ref, out_ref):
    for j in range(POS_PER_STEP):
      x = in_ref[j]
      pc = pos_ref[:, j * 128:j * 128 + 1]
      out_ref[j] = (
          jnp.concatenate([x[:, :embed].T, x[:, embed:].T], axis=1) + pc)

  return pl.pallas_call(
      body,
      grid=(maxlen // POS_PER_STEP,),
      in_specs=[
          pl.BlockSpec((POS_PER_STEP, batch // 2, 2 * embed),
                       lambda l: (l, 0, 0)),
          pl.BlockSpec((embed, POS_PER_STEP * 128), lambda l: (0, l)),
      ],
      out_specs=pl.BlockSpec((POS_PER_STEP, embed, batch),
                             lambda l: (l, 0, 0)),
      out_shape=jax.ShapeDtypeStruct((maxlen, embed, batch), jnp.float32),
      compiler_params=pltpu.CompilerParams(
          dimension_semantics=("parallel",)),
  )(inter3, pos_rep)


@jax.jit
def kernel(x, token_table, pos_table):
  batch, maxlen = x.shape
  embed = token_table.shape[1]
  x_t = x.T.astype(jnp.int32)
  idx_seq = x_t.reshape(maxlen, 2, batch // 2).swapaxes(1, 2).reshape(-1)
  idx_flat = _permute_idx(idx_seq)
  table_lin = _relayout_table(token_table).reshape(-1, embed)
  k = _build_sc_gather(batch, maxlen, embed)
  inter = k(table_lin, idx_flat)
  out_t = _transpose_add(inter, pos_table, batch, maxlen, embed)
  return out_t.transpose(2, 0, 1)
